# baseline (speedup 1.0000x reference)
import functools

import jax
import jax.numpy as jnp
from jax.experimental import pallas as pl
from jax.experimental.pallas import tpu as pltpu


def _fused_kernel(item_idx_ref, x_ref, wih_ref, whh_ref, bx_ref, bhn_ref,
                  item_ref, user_ref, w_out_ref, b_out_ref, out_ref,
                  *, seq_len):
    del item_idx_ref
    H = whh_ref.shape[0]

    xcat = (jnp.dot(x_ref[...], wih_ref[...],
                    preferred_element_type=jnp.float32)
            + bx_ref[...])

    whh = whh_ref[...]
    b_hn = bhn_ref[...]

    h = jnp.zeros((1, H), jnp.float32)
    h_sum = jnp.zeros((1, H), jnp.float32)

    for t in range(seq_len):
        xt = xcat[t:t + 1, :]
        hh = jnp.dot(h.astype(jnp.bfloat16), whh,
                     preferred_element_type=jnp.float32)
        rz = jax.nn.sigmoid(xt[:, :2 * H] + hh[:, :2 * H])
        r = rz[:, :H]
        z = rz[:, H:]
        n = jnp.tanh(xt[:, 2 * H:] + r * (hh[:, 2 * H:] + b_hn))
        h = n + z * (h - n)
        h_sum = h_sum + h

    mean_h = h_sum * (1.0 / float(seq_len))
    scale = item_ref[0] * mean_h

    mul = (user_ref[...] * scale).astype(jnp.bfloat16)
    logits = (jnp.dot(mul, w_out_ref[...],
                      preferred_element_type=jnp.float32)
              + b_out_ref[...])
    m = jnp.max(logits, axis=-1, keepdims=True)
    e = jnp.exp(logits - m)
    out_ref[...] = e / jnp.sum(e, axis=-1, keepdims=True)


def kernel(item_table, user_table, w_ih, w_hh, b_ih, b_hh, w_out, b_out,
           item_id, user_ids, word_embeddings):
    seq_len, batch, emb_dim = word_embeddings.shape
    hidden = w_hh.shape[-1]
    rating_range = w_out.shape[-1]
    assert batch == 1 and hidden == emb_dim

    num_users = user_ids.shape[0]
    item_idx = jnp.reshape(item_id, (1,))

    wih_cat = jnp.transpose(w_ih, (1, 0, 2)).reshape(
        emb_dim, 3 * hidden).astype(jnp.bfloat16)
    whh_cat = jnp.transpose(w_hh, (1, 0, 2)).reshape(
        hidden, 3 * hidden).astype(jnp.bfloat16)
    b_x = jnp.concatenate(
        [b_ih[0] + b_hh[0], b_ih[1] + b_hh[1], b_ih[2]], axis=1)
    b_hn = b_hh[2]

    xb = word_embeddings.reshape(seq_len, emb_dim).astype(jnp.bfloat16)
    w_out_b = w_out.astype(jnp.bfloat16)
    user_emb = user_table[jnp.asarray(user_ids)]

    kern = functools.partial(_fused_kernel, seq_len=seq_len)
    grid_spec = pltpu.PrefetchScalarGridSpec(
        num_scalar_prefetch=1,
        grid=(1,),
        in_specs=[
            pl.BlockSpec((seq_len, emb_dim), lambda i, ii: (0, 0)),
            pl.BlockSpec((emb_dim, 3 * hidden), lambda i, ii: (0, 0)),
            pl.BlockSpec((hidden, 3 * hidden), lambda i, ii: (0, 0)),
            pl.BlockSpec((1, 3 * hidden), lambda i, ii: (0, 0)),
            pl.BlockSpec((1, hidden), lambda i, ii: (0, 0)),
            pl.BlockSpec((1, 1, emb_dim), lambda i, ii: (ii[0], 0, 0)),
            pl.BlockSpec((num_users, emb_dim), lambda i, ii: (0, 0)),
            pl.BlockSpec((hidden, rating_range), lambda i, ii: (0, 0)),
            pl.BlockSpec((1, rating_range), lambda i, ii: (0, 0)),
        ],
        out_specs=pl.BlockSpec((num_users, rating_range),
                               lambda i, ii: (0, 0)),
    )
    return pl.pallas_call(
        kern,
        out_shape=jax.ShapeDtypeStruct((num_users, rating_range),
                                       jnp.float32),
        grid_spec=grid_spec,
        compiler_params=pltpu.CompilerParams(
            dimension_semantics=("arbitrary",)),
    )(item_idx, xb, wih_cat, whh_cat, b_x, b_hn,
      item_table.reshape(item_table.shape[0], 1, emb_dim),
      user_emb, w_out_b, b_out)
